# baseline (speedup 1.0000x reference)
import functools

import jax
import jax.numpy as jnp
from jax import lax
from jax.experimental import pallas as pl
from jax.experimental.pallas import tpu as pltpu
from jax.experimental.pallas import tpu_sc as plsc

B = 2
N = 10000
E = 160000
FEAT = 128
HEADS = 4
HID = 32
OUT_FINAL = 131
DP = 144
NB = B * N

TILES = 16
CHUNK = 80
EPAD = 163840
CPT = EPAD // (TILES * CHUNK)
EPT = CPT * CHUNK
WIN = 5000
NJW = WIN + 16
RPT = 312

_f32 = jnp.float32


def _sc_body(lay2, w, h_hbm, als_hbm, ald_hbm, src_hbm, dst_hbm, out_hbm,
             src_all, dst_all,
             srcw0, dstw0, dstg0, als_v0, ald_v0, h_buf0,
             srcw1, dstw1, dstg1, als_v1, ald_v1, h_buf1,
             sem_a, sem_b, out_sh):
    h_buf = h_buf0
    c = lax.axis_index("c")
    s = lax.axis_index("s")
    nq = DP // 16
    zero16 = jnp.zeros((16,), _f32)
    shift = (c * N).astype(jnp.int32)
    wbase = jnp.int32(w * WIN)
    junk = (WIN + s).astype(jnp.int32)

    pltpu.sync_copy(src_hbm.at[pl.ds(s * EPT, EPT)], src_all)
    pltpu.sync_copy(dst_hbm.at[pl.ds(s * EPT, EPT)], dst_all)

    @plsc.parallel_loop(0, CHUNK, unroll=4)
    def zrow(e):
        for q in range(nq):
            h_buf[e, pl.ds(16 * q, 16)] = zero16
    r0 = s * RPT
    ofs = 0
    while ofs < RPT:
        sz = min(CHUNK, RPT - ofs)
        pltpu.sync_copy(h_buf.at[pl.ds(0, sz)], out_sh.at[pl.ds(r0 + ofs, sz)])
        ofs += sz
    NREM = NJW - TILES * RPT
    @pl.when(s == 0)
    def _():
        pltpu.sync_copy(h_buf.at[pl.ds(0, NREM)],
                        out_sh.at[pl.ds(TILES * RPT, NREM)])
    plsc.subcore_barrier()

    bufs = ((srcw0, dstw0, dstg0, als_v0, ald_v0, h_buf0, sem_a),
            (srcw1, dstw1, dstg1, als_v1, ald_v1, h_buf1, sem_b))

    def stage(k, bb):
        srcw, dstw, dstg, als_v, ald_v, hb, sem = bb

        def mkidx(i, carry):
            g = pl.ds(k * CHUNK + 16 * i, 16)
            srcw[0, pl.ds(16 * i, 16)] = src_all[g] + shift
            dv = dst_all[g]
            dstg[0, pl.ds(16 * i, 16)] = dv + shift
            dl = dv - wbase
            ok = (dl >= 0) & (dl < WIN)
            dstw[0, pl.ds(16 * i, 16)] = jnp.where(ok, dl, junk)
            return carry
        lax.fori_loop(0, CHUNK // 16, mkidx, None)
        pltpu.async_copy(h_hbm.at[srcw.at[0]], hb, sem)
        pltpu.async_copy(als_hbm.at[srcw.at[0]], als_v, sem)
        pltpu.async_copy(ald_hbm.at[dstg.at[0]], ald_v, sem)

    def finish(k, bb):
        srcw, dstw, dstg, als_v, ald_v, hb, sem = bb
        pltpu.make_async_copy(h_hbm.at[srcw.at[0]], hb, sem).wait()
        pltpu.make_async_copy(als_hbm.at[srcw.at[0]], als_v, sem).wait()
        pltpu.make_async_copy(ald_hbm.at[dstg.at[0]], ald_v, sem).wait()

        @plsc.parallel_loop(0, CHUNK, unroll=8)
        def fuse(e):
            v = als_v[e, pl.ds(0, 16)] + ald_v[e, pl.ds(0, 16)]
            exr = jnp.exp(jnp.maximum(v, 0.2 * v))
            if lay2:
                scals = [exr[0]] * HEADS
            else:
                scals = [exr[0], exr[1], exr[2], exr[3]]
            for q in range(nq - 1):
                hb[e, pl.ds(16 * q, 16)] = (
                    hb[e, pl.ds(16 * q, 16)] * scals[q // 2])
            q = nq - 1
            hb[e, pl.ds(16 * q, 16)] = hb[e, pl.ds(16 * q, 16)] * exr

        pltpu.sync_copy(hb, out_sh.at[dstw.at[0]], add=True)

    stage(0, bufs[0])

    def pair_body(i, carry):
        k0 = 2 * i
        stage(k0 + 1, bufs[1])
        finish(k0, bufs[0])
        @pl.when(k0 + 2 < CPT)
        def _():
            stage(k0 + 2, bufs[0])
        finish(k0 + 1, bufs[1])
        return carry

    lax.fori_loop(0, CPT // 2, pair_body, None)
    plsc.subcore_barrier()

    pltpu.sync_copy(out_sh.at[pl.ds(r0, RPT)],
                    out_hbm.at[pl.ds(c * WIN + r0, RPT)])
    NTAIL = WIN - TILES * RPT
    @pl.when(s == 0)
    def _():
        pltpu.sync_copy(out_sh.at[pl.ds(TILES * RPT, NTAIL)],
                        out_hbm.at[pl.ds(c * WIN + TILES * RPT, NTAIL)])


def _make_sc_pass(lay2, w):
    mesh = plsc.VectorSubcoreMesh(core_axis_name="c", subcore_axis_name="s")
    return pl.kernel(
        functools.partial(_sc_body, lay2, w),
        out_type=jax.ShapeDtypeStruct((B * WIN, DP), _f32),
        mesh=mesh,
        compiler_params=pltpu.CompilerParams(use_tc_tiling_on_sc=False),
        scratch_types=(
            [pltpu.VMEM((EPT,), jnp.int32),
             pltpu.VMEM((EPT,), jnp.int32)
             ] +
            [pltpu.VMEM((1, CHUNK), jnp.int32),
             pltpu.VMEM((1, CHUNK), jnp.int32),
             pltpu.VMEM((1, CHUNK), jnp.int32),
             pltpu.VMEM((CHUNK, 16), _f32),
             pltpu.VMEM((CHUNK, 16), _f32),
             pltpu.VMEM((CHUNK, DP), _f32)
             ] * 2 +
            [pltpu.SemaphoreType.DMA,
             pltpu.SemaphoreType.DMA,
             pltpu.VMEM_SHARED((NJW, DP), _f32)]
        ),
    )


def _sc_layer(lay2, h, als, ald, src, dst):
    parts = [_make_sc_pass(lay2, w)(h, als, ald, src, dst) for w in range(2)]
    return jnp.concatenate(
        [p.reshape(B, WIN, DP) for p in parts], axis=1).reshape(NB, DP)


def _leaky(h):
    return jnp.maximum(h, 0.2 * h)


def _style_body(z_ref, w1, b1, w2, b2, w3, b3, wg0s, sv_ref):
    t = _leaky(jnp.dot(z_ref[...], w1[...], preferred_element_type=_f32)
               + b1[...])
    t = _leaky(jnp.dot(t, w2[...], preferred_element_type=_f32) + b2[...])
    t = jnp.dot(t, w3[...], preferred_element_type=_f32) + b3[...]
    sv_ref[...] = jnp.dot(t, wg0s[...], preferred_element_type=_f32)


def _markers(rows, width, heads):
    return jnp.concatenate(
        [jnp.ones((rows, heads), _f32),
         jnp.zeros((rows, width - heads), _f32)], axis=1)


def _stage_a_body(x_ref, wx, sv, a_s, a_d, h_ref, als_ref, ald_ref):
    xw = jnp.dot(x_ref[...], wx[...], preferred_element_type=_f32)
    mk = _markers(xw.shape[0], DP - FEAT, HEADS)
    for b in range(B):
        hb = xw + sv[b, :][None, :]
        h_ref[b] = jnp.concatenate([hb, mk], axis=1)
        als_ref[b] = jnp.dot(hb, a_s[...], preferred_element_type=_f32)
        ald_ref[b] = jnp.dot(hb, a_d[...], preferred_element_type=_f32)


def _stage_b_body(lay2, outsum_ref, h_ref, als_ref, ald_ref,
                  bg, lg, lb, w_next, a_s, a_d, hn_ref, alsn_ref, aldn_ref):
    rows = outsum_ref.shape[0]
    ex_loop = jnp.exp(_leaky(als_ref[:, :HEADS] + ald_ref[:, :HEADS]))
    full = outsum_ref[...]
    rep = FEAT // HEADS
    exw = jnp.broadcast_to(ex_loop[:, :, None], (rows, HEADS, rep)).reshape(
        rows, FEAT)
    num = full[:, :FEAT] + exw * h_ref[:, :FEAT]
    den = full[:, FEAT:FEAT + HEADS] + ex_loop
    denw = jnp.broadcast_to(den[:, :, None], (rows, HEADS, rep)).reshape(
        rows, FEAT)
    agg = num / denw + bg[...]
    mu = agg.mean(-1, keepdims=True)
    var = ((agg - mu) ** 2).mean(-1, keepdims=True)
    hh = _leaky((agg - mu) / jnp.sqrt(var + 1e-5) * lg[...] + lb[...])
    hn = jnp.dot(hh, w_next[...], preferred_element_type=_f32)
    mk = _markers(rows, DP - hn.shape[1], 1 if lay2 else HEADS)
    hn_ref[...] = jnp.concatenate([hn, mk], axis=1)
    alsn_ref[...] = jnp.dot(hn, a_s[...], preferred_element_type=_f32)
    aldn_ref[...] = jnp.dot(hn, a_d[...], preferred_element_type=_f32)


def _stage_c_body(outsum_ref, h_ref, als_ref, ald_ref, x_ref, pos_ref, bg2,
                  out_ref):
    ex_loop = jnp.exp(_leaky(als_ref[:, 0:1] + ald_ref[:, 0:1]))
    num = outsum_ref[:, :OUT_FINAL] + ex_loop * h_ref[:, :OUT_FINAL]
    den = outsum_ref[:, OUT_FINAL:OUT_FINAL + 1] + ex_loop
    hout = num / den + bg2[...]
    out_ref[:, :FEAT] = x_ref[...] + hout[:, 3:OUT_FINAL]
    out_ref[:, FEAT:] = pos_ref[...] + hout[:, 0:3]


def _head_map(a, din):
    heads, outc = a.shape
    if heads == 1:
        m = jnp.broadcast_to(a.reshape(outc, 1), (outc, 16))
        m = jnp.pad(m, ((0, din - outc), (0, 0)))
    else:
        eye = jnp.eye(heads, dtype=_f32)
        m = (a[:, :, None] * eye[:, None, :]).reshape(heads * outc, heads)
        m = jnp.pad(m, ((0, din - heads * outc), (0, 16 - heads)))
    return m


def kernel(z, x, pos, edge_index, W_s1, b_s1, W_s2, b_s2, W_s3, b_s3,
           W_g0, as0, ad0, bg0, ln0_g, ln0_b,
           W_g1, as1, ad1, bg1, ln1_g, ln1_b,
           W_g2, as2, ad2, bg2):
    src = jnp.concatenate([edge_index[0],
                           jnp.zeros((EPAD - E,), jnp.int32)])
    dst = jnp.concatenate([edge_index[1],
                           jnp.full((EPAD - E,), N, jnp.int32)])
    A0s, A0d = _head_map(as0, FEAT), _head_map(ad0, FEAT)
    A1s, A1d = _head_map(as1, FEAT), _head_map(ad1, FEAT)
    A2s, A2d = _head_map(as2, OUT_FINAL), _head_map(ad2, OUT_FINAL)
    Wg2p = W_g2
    bg2p = bg2.reshape(1, OUT_FINAL)

    sv = pl.pallas_call(
        _style_body,
        out_shape=jax.ShapeDtypeStruct((B, FEAT), _f32),
    )(z, W_s1, b_s1.reshape(1, -1), W_s2, b_s2.reshape(1, -1),
      W_s3, b_s3.reshape(1, -1), W_g0[FEAT:, :])

    RA = 1000
    h0, als0, ald0 = pl.pallas_call(
        _stage_a_body,
        grid=(N // RA,),
        in_specs=[
            pl.BlockSpec((RA, FEAT), lambda i: (i, 0)),
            pl.BlockSpec((FEAT, FEAT), lambda i: (0, 0)),
            pl.BlockSpec((B, FEAT), lambda i: (0, 0)),
            pl.BlockSpec((FEAT, 16), lambda i: (0, 0)),
            pl.BlockSpec((FEAT, 16), lambda i: (0, 0)),
        ],
        out_specs=[
            pl.BlockSpec((B, RA, DP), lambda i: (0, i, 0)),
            pl.BlockSpec((B, RA, 16), lambda i: (0, i, 0)),
            pl.BlockSpec((B, RA, 16), lambda i: (0, i, 0)),
        ],
        out_shape=[
            jax.ShapeDtypeStruct((B, N, DP), _f32),
            jax.ShapeDtypeStruct((B, N, 16), _f32),
            jax.ShapeDtypeStruct((B, N, 16), _f32),
        ],
    )(x, W_g0[:FEAT, :], sv, A0s, A0d)
    h0 = h0.reshape(NB, DP)
    als0 = als0.reshape(NB, 16)
    ald0 = ald0.reshape(NB, 16)

    outsum0 = _sc_layer(False, h0, als0, ald0, src, dst)

    RB = 2000
    def stage_b(lay2, outsum, h, als, ald, bg, lg, lb, w, a_s, a_d):
        dn = OUT_FINAL if lay2 else FEAT
        return pl.pallas_call(
            functools.partial(_stage_b_body, lay2),
            grid=(NB // RB,),
            in_specs=[
                pl.BlockSpec((RB, DP), lambda i: (i, 0)),
                pl.BlockSpec((RB, DP), lambda i: (i, 0)),
                pl.BlockSpec((RB, 16), lambda i: (i, 0)),
                pl.BlockSpec((RB, 16), lambda i: (i, 0)),
                pl.BlockSpec((1, FEAT), lambda i: (0, 0)),
                pl.BlockSpec((1, FEAT), lambda i: (0, 0)),
                pl.BlockSpec((1, FEAT), lambda i: (0, 0)),
                pl.BlockSpec((FEAT, dn), lambda i: (0, 0)),
                pl.BlockSpec((dn, 16), lambda i: (0, 0)),
                pl.BlockSpec((dn, 16), lambda i: (0, 0)),
            ],
            out_specs=[
                pl.BlockSpec((RB, DP), lambda i: (i, 0)),
                pl.BlockSpec((RB, 16), lambda i: (i, 0)),
                pl.BlockSpec((RB, 16), lambda i: (i, 0)),
            ],
            out_shape=[
                jax.ShapeDtypeStruct((NB, DP), _f32),
                jax.ShapeDtypeStruct((NB, 16), _f32),
                jax.ShapeDtypeStruct((NB, 16), _f32),
            ],
        )(outsum, h, als, ald, bg.reshape(1, -1),
          lg.reshape(1, -1), lb.reshape(1, -1), w, a_s, a_d)

    h1, als1, ald1 = stage_b(False, outsum0, h0, als0, ald0,
                             bg0, ln0_g, ln0_b, W_g1, A1s, A1d)
    outsum1 = _sc_layer(False, h1, als1, ald1, src, dst)

    h2, als2, ald2 = stage_b(True, outsum1, h1, als1, ald1,
                             bg1, ln1_g, ln1_b, Wg2p, A2s, A2d)
    outsum2 = _sc_layer(True, h2, als2, ald2, src, dst)

    RC = 1000
    out = pl.pallas_call(
        _stage_c_body,
        grid=(B, N // RC),
        in_specs=[
            pl.BlockSpec((RC, DP), lambda b, i: (b * (N // RC) + i, 0)),
            pl.BlockSpec((RC, DP), lambda b, i: (b * (N // RC) + i, 0)),
            pl.BlockSpec((RC, 16), lambda b, i: (b * (N // RC) + i, 0)),
            pl.BlockSpec((RC, 16), lambda b, i: (b * (N // RC) + i, 0)),
            pl.BlockSpec((RC, FEAT), lambda b, i: (i, 0)),
            pl.BlockSpec((RC, 3), lambda b, i: (i, 0)),
            pl.BlockSpec((1, OUT_FINAL), lambda b, i: (0, 0)),
        ],
        out_specs=pl.BlockSpec((RC, OUT_FINAL),
                               lambda b, i: (b * (N // RC) + i, 0)),
        out_shape=jax.ShapeDtypeStruct((NB, OUT_FINAL), _f32),
    )(outsum2, h2, als2, ald2, x, pos, bg2p)
    return out


# speedup vs baseline: 1.0039x; 1.0039x over previous
import functools

import jax
import jax.numpy as jnp
from jax import lax
from jax.experimental import pallas as pl
from jax.experimental.pallas import tpu as pltpu
from jax.experimental.pallas import tpu_sc as plsc

B = 2
N = 10000
E = 160000
FEAT = 128
HEADS = 4
HID = 32
OUT_FINAL = 131
DP = 144
NB = B * N

TILES = 16
CHUNK = 80
EPAD = 163840
CPT = EPAD // (TILES * CHUNK)
EPT = CPT * CHUNK
WIN = 5000
NJW = WIN + 16
RPT = 312

_f32 = jnp.float32


def _sc_body(lay2, w, h_hbm, als_hbm, ald_hbm, src_hbm, dst_hbm, out_hbm,
             src_all, dst_all,
             srcw0, dstw0, dstg0, als_v0, ald_v0, h_buf0,
             srcw1, dstw1, dstg1, als_v1, ald_v1, h_buf1,
             sem_a, sem_b, out_sh):
    h_buf = h_buf0
    c = lax.axis_index("c")
    s = lax.axis_index("s")
    nq = DP // 16
    zero16 = jnp.zeros((16,), _f32)
    shift = (c * N).astype(jnp.int32)
    wbase = jnp.int32(w * WIN)
    junk = (WIN + s).astype(jnp.int32)

    pltpu.sync_copy(src_hbm.at[pl.ds(s * EPT, EPT)], src_all)
    pltpu.sync_copy(dst_hbm.at[pl.ds(s * EPT, EPT)], dst_all)

    @plsc.parallel_loop(0, CHUNK, unroll=4)
    def zrow(e):
        for q in range(nq):
            h_buf[e, pl.ds(16 * q, 16)] = zero16
    r0 = s * RPT
    ofs = 0
    while ofs < RPT:
        sz = min(CHUNK, RPT - ofs)
        pltpu.sync_copy(h_buf.at[pl.ds(0, sz)], out_sh.at[pl.ds(r0 + ofs, sz)])
        ofs += sz
    NREM = NJW - TILES * RPT
    @pl.when(s == 0)
    def _():
        pltpu.sync_copy(h_buf.at[pl.ds(0, NREM)],
                        out_sh.at[pl.ds(TILES * RPT, NREM)])
    plsc.subcore_barrier()

    bufs = ((srcw0, dstw0, dstg0, als_v0, ald_v0, h_buf0, sem_a),
            (srcw1, dstw1, dstg1, als_v1, ald_v1, h_buf1, sem_b))

    def stage(k, bb):
        srcw, dstw, dstg, als_v, ald_v, hb, sem = bb

        def mkidx(i, carry):
            g = pl.ds(k * CHUNK + 16 * i, 16)
            srcw[0, pl.ds(16 * i, 16)] = src_all[g] + shift
            dv = dst_all[g]
            dstg[0, pl.ds(16 * i, 16)] = dv + shift
            dl = dv - wbase
            ok = (dl >= 0) & (dl < WIN)
            dstw[0, pl.ds(16 * i, 16)] = jnp.where(ok, dl, junk)
            return carry
        lax.fori_loop(0, CHUNK // 16, mkidx, None)
        pltpu.async_copy(h_hbm.at[srcw.at[0]], hb, sem)
        pltpu.async_copy(als_hbm.at[srcw.at[0]], als_v, sem)
        pltpu.async_copy(ald_hbm.at[dstg.at[0]], ald_v, sem)

    def finish(k, bb):
        srcw, dstw, dstg, als_v, ald_v, hb, sem = bb
        pltpu.make_async_copy(h_hbm.at[srcw.at[0]], hb, sem).wait()
        pltpu.make_async_copy(als_hbm.at[srcw.at[0]], als_v, sem).wait()
        pltpu.make_async_copy(ald_hbm.at[dstg.at[0]], ald_v, sem).wait()

        @plsc.parallel_loop(0, CHUNK, unroll=4)
        def fuse(e):
            v = als_v[e, pl.ds(0, 16)] + ald_v[e, pl.ds(0, 16)]
            exr = jnp.exp(jnp.maximum(v, 0.2 * v))
            if lay2:
                scals = [exr[0]] * HEADS
            else:
                scals = [exr[0], exr[1], exr[2], exr[3]]
            for q in range(nq - 1):
                hb[e, pl.ds(16 * q, 16)] = (
                    hb[e, pl.ds(16 * q, 16)] * scals[q // 2])
            q = nq - 1
            hb[e, pl.ds(16 * q, 16)] = hb[e, pl.ds(16 * q, 16)] * exr

        pltpu.sync_copy(hb, out_sh.at[dstw.at[0]], add=True)

    stage(0, bufs[0])

    def pair_body(i, carry):
        k0 = 2 * i
        stage(k0 + 1, bufs[1])
        finish(k0, bufs[0])
        @pl.when(k0 + 2 < CPT)
        def _():
            stage(k0 + 2, bufs[0])
        finish(k0 + 1, bufs[1])
        return carry

    lax.fori_loop(0, CPT // 2, pair_body, None)
    plsc.subcore_barrier()

    pltpu.sync_copy(out_sh.at[pl.ds(r0, RPT)],
                    out_hbm.at[pl.ds(c * WIN + r0, RPT)])
    NTAIL = WIN - TILES * RPT
    @pl.when(s == 0)
    def _():
        pltpu.sync_copy(out_sh.at[pl.ds(TILES * RPT, NTAIL)],
                        out_hbm.at[pl.ds(c * WIN + TILES * RPT, NTAIL)])


def _make_sc_pass(lay2, w):
    mesh = plsc.VectorSubcoreMesh(core_axis_name="c", subcore_axis_name="s")
    return pl.kernel(
        functools.partial(_sc_body, lay2, w),
        out_type=jax.ShapeDtypeStruct((B * WIN, DP), _f32),
        mesh=mesh,
        compiler_params=pltpu.CompilerParams(use_tc_tiling_on_sc=False),
        scratch_types=(
            [pltpu.VMEM((EPT,), jnp.int32),
             pltpu.VMEM((EPT,), jnp.int32)
             ] +
            [pltpu.VMEM((1, CHUNK), jnp.int32),
             pltpu.VMEM((1, CHUNK), jnp.int32),
             pltpu.VMEM((1, CHUNK), jnp.int32),
             pltpu.VMEM((CHUNK, 16), _f32),
             pltpu.VMEM((CHUNK, 16), _f32),
             pltpu.VMEM((CHUNK, DP), _f32)
             ] * 2 +
            [pltpu.SemaphoreType.DMA,
             pltpu.SemaphoreType.DMA,
             pltpu.VMEM_SHARED((NJW, DP), _f32)]
        ),
    )


def _sc_layer(lay2, h, als, ald, src, dst):
    parts = [_make_sc_pass(lay2, w)(h, als, ald, src, dst) for w in range(2)]
    return jnp.concatenate(
        [p.reshape(B, WIN, DP) for p in parts], axis=1).reshape(NB, DP)


def _leaky(h):
    return jnp.maximum(h, 0.2 * h)


def _style_body(z_ref, w1, b1, w2, b2, w3, b3, wg0s, sv_ref):
    t = _leaky(jnp.dot(z_ref[...], w1[...], preferred_element_type=_f32)
               + b1[...])
    t = _leaky(jnp.dot(t, w2[...], preferred_element_type=_f32) + b2[...])
    t = jnp.dot(t, w3[...], preferred_element_type=_f32) + b3[...]
    sv_ref[...] = jnp.dot(t, wg0s[...], preferred_element_type=_f32)


def _markers(rows, width, heads):
    return jnp.concatenate(
        [jnp.ones((rows, heads), _f32),
         jnp.zeros((rows, width - heads), _f32)], axis=1)


def _stage_a_body(x_ref, wx, sv, a_s, a_d, h_ref, als_ref, ald_ref):
    xw = jnp.dot(x_ref[...], wx[...], preferred_element_type=_f32)
    mk = _markers(xw.shape[0], DP - FEAT, HEADS)
    for b in range(B):
        hb = xw + sv[b, :][None, :]
        h_ref[b] = jnp.concatenate([hb, mk], axis=1)
        als_ref[b] = jnp.dot(hb, a_s[...], preferred_element_type=_f32)
        ald_ref[b] = jnp.dot(hb, a_d[...], preferred_element_type=_f32)


def _stage_b_body(lay2, outsum_ref, h_ref, als_ref, ald_ref,
                  bg, lg, lb, w_next, a_s, a_d, hn_ref, alsn_ref, aldn_ref):
    rows = outsum_ref.shape[0]
    ex_loop = jnp.exp(_leaky(als_ref[:, :HEADS] + ald_ref[:, :HEADS]))
    full = outsum_ref[...]
    rep = FEAT // HEADS
    exw = jnp.broadcast_to(ex_loop[:, :, None], (rows, HEADS, rep)).reshape(
        rows, FEAT)
    num = full[:, :FEAT] + exw * h_ref[:, :FEAT]
    den = full[:, FEAT:FEAT + HEADS] + ex_loop
    denw = jnp.broadcast_to(den[:, :, None], (rows, HEADS, rep)).reshape(
        rows, FEAT)
    agg = num / denw + bg[...]
    mu = agg.mean(-1, keepdims=True)
    var = ((agg - mu) ** 2).mean(-1, keepdims=True)
    hh = _leaky((agg - mu) / jnp.sqrt(var + 1e-5) * lg[...] + lb[...])
    hn = jnp.dot(hh, w_next[...], preferred_element_type=_f32)
    mk = _markers(rows, DP - hn.shape[1], 1 if lay2 else HEADS)
    hn_ref[...] = jnp.concatenate([hn, mk], axis=1)
    alsn_ref[...] = jnp.dot(hn, a_s[...], preferred_element_type=_f32)
    aldn_ref[...] = jnp.dot(hn, a_d[...], preferred_element_type=_f32)


def _stage_c_body(outsum_ref, h_ref, als_ref, ald_ref, x_ref, pos_ref, bg2,
                  out_ref):
    ex_loop = jnp.exp(_leaky(als_ref[:, 0:1] + ald_ref[:, 0:1]))
    num = outsum_ref[:, :OUT_FINAL] + ex_loop * h_ref[:, :OUT_FINAL]
    den = outsum_ref[:, OUT_FINAL:OUT_FINAL + 1] + ex_loop
    hout = num / den + bg2[...]
    out_ref[:, :FEAT] = x_ref[...] + hout[:, 3:OUT_FINAL]
    out_ref[:, FEAT:] = pos_ref[...] + hout[:, 0:3]


def _head_map(a, din):
    heads, outc = a.shape
    if heads == 1:
        m = jnp.broadcast_to(a.reshape(outc, 1), (outc, 16))
        m = jnp.pad(m, ((0, din - outc), (0, 0)))
    else:
        eye = jnp.eye(heads, dtype=_f32)
        m = (a[:, :, None] * eye[:, None, :]).reshape(heads * outc, heads)
        m = jnp.pad(m, ((0, din - heads * outc), (0, 16 - heads)))
    return m


def kernel(z, x, pos, edge_index, W_s1, b_s1, W_s2, b_s2, W_s3, b_s3,
           W_g0, as0, ad0, bg0, ln0_g, ln0_b,
           W_g1, as1, ad1, bg1, ln1_g, ln1_b,
           W_g2, as2, ad2, bg2):
    src = jnp.concatenate([edge_index[0],
                           jnp.zeros((EPAD - E,), jnp.int32)])
    dst = jnp.concatenate([edge_index[1],
                           jnp.full((EPAD - E,), N, jnp.int32)])
    A0s, A0d = _head_map(as0, FEAT), _head_map(ad0, FEAT)
    A1s, A1d = _head_map(as1, FEAT), _head_map(ad1, FEAT)
    A2s, A2d = _head_map(as2, OUT_FINAL), _head_map(ad2, OUT_FINAL)
    Wg2p = W_g2
    bg2p = bg2.reshape(1, OUT_FINAL)

    sv = pl.pallas_call(
        _style_body,
        out_shape=jax.ShapeDtypeStruct((B, FEAT), _f32),
    )(z, W_s1, b_s1.reshape(1, -1), W_s2, b_s2.reshape(1, -1),
      W_s3, b_s3.reshape(1, -1), W_g0[FEAT:, :])

    RA = 1000
    h0, als0, ald0 = pl.pallas_call(
        _stage_a_body,
        grid=(N // RA,),
        in_specs=[
            pl.BlockSpec((RA, FEAT), lambda i: (i, 0)),
            pl.BlockSpec((FEAT, FEAT), lambda i: (0, 0)),
            pl.BlockSpec((B, FEAT), lambda i: (0, 0)),
            pl.BlockSpec((FEAT, 16), lambda i: (0, 0)),
            pl.BlockSpec((FEAT, 16), lambda i: (0, 0)),
        ],
        out_specs=[
            pl.BlockSpec((B, RA, DP), lambda i: (0, i, 0)),
            pl.BlockSpec((B, RA, 16), lambda i: (0, i, 0)),
            pl.BlockSpec((B, RA, 16), lambda i: (0, i, 0)),
        ],
        out_shape=[
            jax.ShapeDtypeStruct((B, N, DP), _f32),
            jax.ShapeDtypeStruct((B, N, 16), _f32),
            jax.ShapeDtypeStruct((B, N, 16), _f32),
        ],
    )(x, W_g0[:FEAT, :], sv, A0s, A0d)
    h0 = h0.reshape(NB, DP)
    als0 = als0.reshape(NB, 16)
    ald0 = ald0.reshape(NB, 16)

    outsum0 = _sc_layer(False, h0, als0, ald0, src, dst)

    RB = 2000
    def stage_b(lay2, outsum, h, als, ald, bg, lg, lb, w, a_s, a_d):
        dn = OUT_FINAL if lay2 else FEAT
        return pl.pallas_call(
            functools.partial(_stage_b_body, lay2),
            grid=(NB // RB,),
            in_specs=[
                pl.BlockSpec((RB, DP), lambda i: (i, 0)),
                pl.BlockSpec((RB, DP), lambda i: (i, 0)),
                pl.BlockSpec((RB, 16), lambda i: (i, 0)),
                pl.BlockSpec((RB, 16), lambda i: (i, 0)),
                pl.BlockSpec((1, FEAT), lambda i: (0, 0)),
                pl.BlockSpec((1, FEAT), lambda i: (0, 0)),
                pl.BlockSpec((1, FEAT), lambda i: (0, 0)),
                pl.BlockSpec((FEAT, dn), lambda i: (0, 0)),
                pl.BlockSpec((dn, 16), lambda i: (0, 0)),
                pl.BlockSpec((dn, 16), lambda i: (0, 0)),
            ],
            out_specs=[
                pl.BlockSpec((RB, DP), lambda i: (i, 0)),
                pl.BlockSpec((RB, 16), lambda i: (i, 0)),
                pl.BlockSpec((RB, 16), lambda i: (i, 0)),
            ],
            out_shape=[
                jax.ShapeDtypeStruct((NB, DP), _f32),
                jax.ShapeDtypeStruct((NB, 16), _f32),
                jax.ShapeDtypeStruct((NB, 16), _f32),
            ],
        )(outsum, h, als, ald, bg.reshape(1, -1),
          lg.reshape(1, -1), lb.reshape(1, -1), w, a_s, a_d)

    h1, als1, ald1 = stage_b(False, outsum0, h0, als0, ald0,
                             bg0, ln0_g, ln0_b, W_g1, A1s, A1d)
    outsum1 = _sc_layer(False, h1, als1, ald1, src, dst)

    h2, als2, ald2 = stage_b(True, outsum1, h1, als1, ald1,
                             bg1, ln1_g, ln1_b, Wg2p, A2s, A2d)
    outsum2 = _sc_layer(True, h2, als2, ald2, src, dst)

    RC = 1000
    out = pl.pallas_call(
        _stage_c_body,
        grid=(B, N // RC),
        in_specs=[
            pl.BlockSpec((RC, DP), lambda b, i: (b * (N // RC) + i, 0)),
            pl.BlockSpec((RC, DP), lambda b, i: (b * (N // RC) + i, 0)),
            pl.BlockSpec((RC, 16), lambda b, i: (b * (N // RC) + i, 0)),
            pl.BlockSpec((RC, 16), lambda b, i: (b * (N // RC) + i, 0)),
            pl.BlockSpec((RC, FEAT), lambda b, i: (i, 0)),
            pl.BlockSpec((RC, 3), lambda b, i: (i, 0)),
            pl.BlockSpec((1, OUT_FINAL), lambda b, i: (0, 0)),
        ],
        out_specs=pl.BlockSpec((RC, OUT_FINAL),
                               lambda b, i: (b * (N // RC) + i, 0)),
        out_shape=jax.ShapeDtypeStruct((NB, OUT_FINAL), _f32),
    )(outsum2, h2, als2, ald2, x, pos, bg2p)
    return out
